# baseline (speedup 1.0000x reference)
import functools

import jax
import jax.numpy as jnp
from jax import lax
from jax.experimental import pallas as pl
from jax.experimental.pallas import tpu as pltpu
from jax.experimental.pallas import tpu_sc as plsc

_D_MODEL = 4096
_NUM_EXPERTS = 64
_TOP_K = 8
_BLOCK_T = 1024
_N_TOKENS = 32768

_NW = 32
_ROWS_PER_W = _N_TOKENS // _NW
_CHUNK = 128


def _gate_tc_kernel(x_ref, w_ref, probs_ref, logits_ref):
    logits_t = jax.lax.dot_general(
        w_ref[...], x_ref[...],
        dimension_numbers=(((1,), (1,)), ((), ())),
        preferred_element_type=jnp.float32,
    )
    logits_ref[...] = logits_t.T
    m = jnp.max(logits_t, axis=0, keepdims=True)
    e = jnp.exp(logits_t - m)
    s = jnp.sum(e, axis=0, keepdims=True)
    probs_ref[...] = (e / s).T


def _sc_topk_body(probs_hbm, outw_hbm, outi_hbm, buf, ow, oi):
    c_idx = lax.axis_index("c")
    s_idx = lax.axis_index("s")
    wid = s_idx * 2 + c_idx
    base = wid * _ROWS_PER_W
    lane = lax.iota(jnp.int32, 16)
    lane64 = lane * _NUM_EXPERTS
    lane8 = lane * _TOP_K

    def do_chunk(c, carry0):
        t0 = base + c * _CHUNK
        pltpu.sync_copy(
            probs_hbm.at[pl.ds(t0 * _NUM_EXPERTS, _CHUNK * _NUM_EXPERTS)],
            buf,
        )

        def do_group(g, carry1):

            def do_expert(e_i, carry):
                ws, idxs, psum = carry
                gidx = lane64 + (g * (16 * _NUM_EXPERTS) + e_i)
                v = plsc.load_gather(buf, [gidx])
                psum = psum + v
                vi = jnp.full((16,), e_i, jnp.int32)
                new_ws = []
                new_is = []
                for k in range(_TOP_K):
                    gt = v > ws[k]
                    new_ws.append(jnp.where(gt, v, ws[k]))
                    new_is.append(jnp.where(gt, vi, idxs[k]))
                    v = jnp.where(gt, ws[k], v)
                    vi = jnp.where(gt, idxs[k], vi)
                return (tuple(new_ws), tuple(new_is), psum)

            neg = jnp.full((16,), -1.0, jnp.float32)
            zero_i = jnp.full((16,), 0, jnp.int32)
            ws, idxs, psum = lax.fori_loop(
                0, _NUM_EXPERTS, do_expert,
                ((neg,) * _TOP_K, (zero_i,) * _TOP_K,
                 jnp.full((16,), 0.0, jnp.float32)),
            )
            rcp = 1.0 / psum
            for k in range(_TOP_K):
                sidx = lane8 + (g * (16 * _TOP_K) + k)
                plsc.store_scatter(ow, [sidx], ws[k] * rcp)
                plsc.store_scatter(oi, [sidx], idxs[k])
            return carry1

        lax.fori_loop(0, _CHUNK // 16, do_group, 0)
        pltpu.sync_copy(
            ow, outw_hbm.at[pl.ds(t0 * _TOP_K, _CHUNK * _TOP_K)]
        )
        pltpu.sync_copy(
            oi, outi_hbm.at[pl.ds(t0 * _TOP_K, _CHUNK * _TOP_K)]
        )
        return carry0

    lax.fori_loop(0, _ROWS_PER_W // _CHUNK, do_chunk, 0)


_sc_topk = functools.partial(
    pl.kernel,
    out_type=[
        jax.ShapeDtypeStruct((_N_TOKENS * _TOP_K,), jnp.float32),
        jax.ShapeDtypeStruct((_N_TOKENS * _TOP_K,), jnp.int32),
    ],
    mesh=plsc.VectorSubcoreMesh(core_axis_name="c", subcore_axis_name="s"),
    compiler_params=pltpu.CompilerParams(needs_layout_passes=False),
    scratch_types=[
        pltpu.VMEM((_CHUNK * _NUM_EXPERTS,), jnp.float32),
        pltpu.VMEM((_CHUNK * _TOP_K,), jnp.float32),
        pltpu.VMEM((_CHUNK * _TOP_K,), jnp.int32),
    ],
)(_sc_topk_body)


@functools.partial(jax.jit, static_argnames=())
def kernel(x, W):
    n_tokens, d_model = x.shape
    n_experts = W.shape[0]
    grid = (n_tokens // _BLOCK_T,)
    probs, logits = pl.pallas_call(
        _gate_tc_kernel,
        grid=grid,
        in_specs=[
            pl.BlockSpec((_BLOCK_T, d_model), lambda i: (i, 0)),
            pl.BlockSpec((n_experts, d_model), lambda i: (0, 0)),
        ],
        out_specs=[
            pl.BlockSpec((_BLOCK_T, n_experts), lambda i: (i, 0)),
            pl.BlockSpec((_BLOCK_T, n_experts), lambda i: (i, 0)),
        ],
        out_shape=[
            jax.ShapeDtypeStruct((n_tokens, n_experts), jnp.float32),
            jax.ShapeDtypeStruct((n_tokens, n_experts), jnp.float32),
        ],
        compiler_params=pltpu.CompilerParams(
            dimension_semantics=("arbitrary",),
        ),
    )(x, W)
    topk_w_flat, topk_i_flat = _sc_topk(probs.reshape(-1))
    topk_w = topk_w_flat.reshape(n_tokens, _TOP_K)
    topk_i = topk_i_flat.reshape(n_tokens, _TOP_K)
    return (topk_w, probs, topk_i, logits)


# speedup vs baseline: 1.4275x; 1.4275x over previous
import functools

import jax
import jax.numpy as jnp
from jax.experimental import pallas as pl
from jax.experimental.pallas import tpu as pltpu

_D_MODEL = 4096
_NUM_EXPERTS = 64
_TOP_K = 8
_BLOCK_T = 1024


def _gate_kernel(x_ref, w_ref, topk_w_ref, probs_ref, topk_i_ref, logits_ref):
    logits_t = jax.lax.dot_general(
        w_ref[...], x_ref[...],
        dimension_numbers=(((1,), (1,)), ((), ())),
        preferred_element_type=jnp.float32,
    )
    logits_ref[...] = logits_t.T

    m = jnp.max(logits_t, axis=0, keepdims=True)
    e = jnp.exp(logits_t - m)
    s = jnp.sum(e, axis=0, keepdims=True)
    probs_t = e / s
    probs_ref[...] = probs_t.T
    psum = jnp.sum(probs_t, axis=0, keepdims=True)

    eiota = jax.lax.broadcasted_iota(jnp.int32, probs_t.shape, 0).astype(
        jnp.float32
    )
    cur = probs_t
    mxs = []
    idxs = []
    for k in range(_TOP_K):
        mx = jnp.max(cur, axis=0, keepdims=True)
        idxf = jnp.min(
            jnp.where(cur == mx, eiota, float(_NUM_EXPERTS)),
            axis=0, keepdims=True,
        )
        mxs.append(mx / psum)
        idxs.append(idxf)
        if k + 1 < _TOP_K:
            cur = jnp.where(eiota == idxf, -1.0, cur)
    topk_w_t = jnp.concatenate(mxs, axis=0)
    topk_i_t = jnp.concatenate(idxs, axis=0)
    topk_w_ref[...] = topk_w_t.T
    topk_i_ref[...] = topk_i_t.T.astype(jnp.int32)


@functools.partial(jax.jit, static_argnames=())
def kernel(x, W):
    n_tokens, d_model = x.shape
    n_experts = W.shape[0]
    grid = (n_tokens // _BLOCK_T,)
    out = pl.pallas_call(
        _gate_kernel,
        grid=grid,
        in_specs=[
            pl.BlockSpec((_BLOCK_T, d_model), lambda i: (i, 0)),
            pl.BlockSpec((n_experts, d_model), lambda i: (0, 0)),
        ],
        out_specs=[
            pl.BlockSpec((_BLOCK_T, _TOP_K), lambda i: (i, 0)),
            pl.BlockSpec((_BLOCK_T, n_experts), lambda i: (i, 0)),
            pl.BlockSpec((_BLOCK_T, _TOP_K), lambda i: (i, 0)),
            pl.BlockSpec((_BLOCK_T, n_experts), lambda i: (i, 0)),
        ],
        out_shape=[
            jax.ShapeDtypeStruct((n_tokens, _TOP_K), jnp.float32),
            jax.ShapeDtypeStruct((n_tokens, n_experts), jnp.float32),
            jax.ShapeDtypeStruct((n_tokens, _TOP_K), jnp.int32),
            jax.ShapeDtypeStruct((n_tokens, n_experts), jnp.float32),
        ],
        compiler_params=pltpu.CompilerParams(
            dimension_semantics=("arbitrary",),
        ),
    )(x, W)
    topk_w, probs, topk_i, logits = out
    return (topk_w, probs, topk_i, logits)


# speedup vs baseline: 1.4391x; 1.0082x over previous
import functools

import jax
import jax.numpy as jnp
from jax.experimental import pallas as pl
from jax.experimental.pallas import tpu as pltpu

_D_MODEL = 4096
_NUM_EXPERTS = 64
_TOP_K = 8
_BT = 512


def _gate_half(x_blk, w):
    logits_t = jax.lax.dot_general(
        w, x_blk,
        dimension_numbers=(((1,), (1,)), ((), ())),
        preferred_element_type=jnp.float32,
    )
    m = jnp.max(logits_t, axis=0, keepdims=True)
    e = jnp.exp(logits_t - m)
    s = jnp.sum(e, axis=0, keepdims=True)
    probs_t = e / s
    psum = jnp.sum(probs_t, axis=0, keepdims=True)

    eiota = jax.lax.broadcasted_iota(jnp.int32, probs_t.shape, 0).astype(
        jnp.float32
    )
    cur = probs_t
    mxs = []
    idxs = []
    for k in range(_TOP_K):
        mx = jnp.max(cur, axis=0, keepdims=True)
        idxf = jnp.min(
            jnp.where(cur == mx, eiota, float(_NUM_EXPERTS)),
            axis=0, keepdims=True,
        )
        mxs.append(mx / psum)
        idxs.append(idxf)
        if k + 1 < _TOP_K:
            cur = jnp.where(eiota == idxf, -1.0, cur)
    topk_w_t = jnp.concatenate(mxs, axis=0)
    topk_i_t = jnp.concatenate(idxs, axis=0)
    return (topk_w_t.T, probs_t.T, topk_i_t.T.astype(jnp.int32), logits_t.T)


def _gate_kernel(x1_ref, x2_ref, w_ref, topk_w_ref, probs_ref, topk_i_ref,
                 logits_ref):
    w = w_ref[...]
    for h, x_ref in enumerate((x1_ref, x2_ref)):
        topk_w, probs, topk_i, logits = _gate_half(x_ref[0, 0], w)
        topk_w_ref[h] = topk_w
        probs_ref[h] = probs
        topk_i_ref[h] = topk_i
        logits_ref[h] = logits


@functools.partial(jax.jit, static_argnames=())
def kernel(x, W):
    n_tokens, d_model = x.shape
    n_experts = W.shape[0]
    half = n_tokens // 2
    half_blocks = half // _BT
    xr = x.reshape(2, half_blocks, _BT, d_model)
    grid = (half_blocks,)
    out = pl.pallas_call(
        _gate_kernel,
        grid=grid,
        in_specs=[
            pl.BlockSpec((1, 1, _BT, d_model), lambda i: (0, i, 0, 0)),
            pl.BlockSpec((1, 1, _BT, d_model), lambda i: (1, i, 0, 0)),
            pl.BlockSpec((n_experts, d_model), lambda i: (0, 0)),
        ],
        out_specs=[
            pl.BlockSpec((2, _BT, _TOP_K), lambda i: (0, i, 0)),
            pl.BlockSpec((2, _BT, _NUM_EXPERTS), lambda i: (0, i, 0)),
            pl.BlockSpec((2, _BT, _TOP_K), lambda i: (0, i, 0)),
            pl.BlockSpec((2, _BT, _NUM_EXPERTS), lambda i: (0, i, 0)),
        ],
        out_shape=[
            jax.ShapeDtypeStruct((2, half, _TOP_K), jnp.float32),
            jax.ShapeDtypeStruct((2, half, _NUM_EXPERTS), jnp.float32),
            jax.ShapeDtypeStruct((2, half, _TOP_K), jnp.int32),
            jax.ShapeDtypeStruct((2, half, _NUM_EXPERTS), jnp.float32),
        ],
        compiler_params=pltpu.CompilerParams(
            dimension_semantics=("arbitrary",),
        ),
    )(xr, xr, W)
    topk_w, probs, topk_i, logits = out
    return (
        topk_w.reshape(n_tokens, _TOP_K),
        probs.reshape(n_tokens, _NUM_EXPERTS),
        topk_i.reshape(n_tokens, _TOP_K),
        logits.reshape(n_tokens, _NUM_EXPERTS),
    )
